# baseline (speedup 1.0000x reference)
import functools

import jax
import jax.numpy as jnp
from jax import lax
from jax.experimental import pallas as pl
from jax.experimental.pallas import tpu as pltpu
from jax.experimental.pallas import tpu_sc as plsc

_N = 10000
_E = 320000
_D = 128
_DL = 512
_B = 2000
_NB = _N // _B
_L = 2

_NC = 2
_NS = 16
_NW = _NC * _NS
_EPT = _E // _NW
_KC = 128
_NCH3 = 27
_NCH = _NCH3 * 3 + 3
_EPT_PAD = _NCH * _KC
_KD = 256
_NCHD = 40
_NP = _N + 8
_SLAB = 624
_REM = _N - _NS * _SLAB

_DEGW = 16

_ROWBLK = 1000
_G = _N // _ROWBLK


def _sc_mesh():
    return plsc.VectorSubcoreMesh(core_axis_name="c", subcore_axis_name="s")


def _copy_slab(si, src_ref, dst_ref):
    pltpu.sync_copy(src_ref.at[pl.ds(si * _SLAB, _SLAB)],
                    dst_ref.at[pl.ds(si * _SLAB, _SLAB)])

    @pl.when(si == _NS - 1)
    def _():
        pltpu.sync_copy(src_ref.at[pl.ds(_NS * _SLAB, _REM)],
                        dst_ref.at[pl.ds(_NS * _SLAB, _REM)])


def _degrees_sc(src_deg, dst_pad, zeros_nd, ones_s, ones_d):
    @functools.partial(
        pl.kernel,
        out_type=jax.ShapeDtypeStruct((_NC, _N, _D), jnp.float32),
        mesh=_sc_mesh(),
        scratch_types=[
            pltpu.VMEM((_KD,), jnp.int32),
            pltpu.VMEM((_KD,), jnp.int32),
            pltpu.VMEM((_KD, _D), jnp.float32),
            pltpu.VMEM_SHARED((_NP, _D), jnp.float32),
            pltpu.SemaphoreType.DMA,
            pltpu.SemaphoreType.DMA,
        ],
    )
    def k(src_hbm, dst_hbm, z_hbm, ones_s_hbm, ones_d_hbm, out_hbm, idx_a,
          idx_b, ones_v, acc, sem_a, sem_b):
        ci = lax.axis_index("c")
        si = lax.axis_index("s")
        wid = si * _NC + ci
        _copy_slab(si, z_hbm, acc)

        def phase(e_hbm, carry0):
            def body(i, carry):
                g0 = i * 2
                g1 = g0 + 1
                cp_a = pltpu.async_copy(e_hbm.at[wid, g0], idx_a, sem_a)
                cp_b = pltpu.async_copy(e_hbm.at[wid, g1], idx_b, sem_b)
                cp_a.wait()
                pltpu.sync_copy(ones_v, acc.at[idx_a], add=True)
                cp_b.wait()
                pltpu.sync_copy(ones_v, acc.at[idx_b], add=True)
                return carry

            lax.fori_loop(0, _NCHD // 2, body, carry0)

        pltpu.sync_copy(ones_s_hbm, ones_v)
        plsc.subcore_barrier()
        phase(src_hbm, 0)
        pltpu.sync_copy(ones_d_hbm, ones_v)
        phase(dst_hbm, 0)
        plsc.subcore_barrier()
        _copy_slab(si, acc, out_hbm.at[ci])

    return k(src_deg, dst_pad, zeros_nd, ones_s, ones_d)


def _spmm_sc(hs, src_g, dst_s, zeros_nd):
    @functools.partial(
        pl.kernel,
        out_type=jax.ShapeDtypeStruct((_NC, _N, _D), jnp.float32),
        mesh=_sc_mesh(),
        scratch_types=[
            [pltpu.VMEM((_KC,), jnp.int32)] * 3,
            [pltpu.VMEM((_KC,), jnp.int32)] * 3,
            [pltpu.VMEM((_KC, _D), jnp.float32)] * 3,
            pltpu.VMEM_SHARED((_NP, _D), jnp.float32),
            [pltpu.SemaphoreType.DMA] * 3,
            [pltpu.SemaphoreType.DMA] * 3,
            [pltpu.SemaphoreType.DMA] * 3,
            [pltpu.SemaphoreType.DMA] * 3,
        ],
    )
    def k(hs_hbm, src_hbm, dst_hbm, z_hbm, out_hbm, sidx, didx, rows, acc,
          sem_s, sem_d, sem_g, sem_w):
        ci = lax.axis_index("c")
        si = lax.axis_index("s")
        wid = si * _NC + ci
        _copy_slab(si, z_hbm, acc)

        def _scat_wait(j):
            pltpu.make_async_copy(rows[j], acc.at[didx[j]], sem_w[j]).wait()

        def _sidx_wait(j, c):
            pltpu.make_async_copy(src_hbm.at[wid, c], sidx[j],
                                  sem_s[j]).wait()

        def _didx_wait(j, c):
            pltpu.make_async_copy(dst_hbm.at[wid, c], didx[j],
                                  sem_d[j]).wait()

        for j in range(3):
            pltpu.async_copy(src_hbm.at[wid, j], sidx[j], sem_s[j])
            pltpu.async_copy(dst_hbm.at[wid, j], didx[j], sem_d[j])
        plsc.subcore_barrier()

        def body(i, carry):
            cbase = i * 3

            @pl.when(i > 0)
            def _():
                for j in range(3):
                    _scat_wait(j)
                    pltpu.async_copy(dst_hbm.at[wid, cbase + j], didx[j],
                                     sem_d[j])

            gaths = []
            for j in range(3):
                _sidx_wait(j, cbase + j)
                gaths.append(
                    pltpu.async_copy(hs_hbm.at[sidx[j]], rows[j], sem_g[j]))
            for j in range(3):
                gaths[j].wait()
                pltpu.async_copy(src_hbm.at[wid, cbase + j + 3], sidx[j],
                                 sem_s[j])
                _didx_wait(j, cbase + j)
                pltpu.async_copy(rows[j], acc.at[didx[j]], sem_w[j],
                                 add=True)
            return carry

        lax.fori_loop(0, _NCH3, body, 0)
        for j in range(3):
            _scat_wait(j)
            _sidx_wait(j, _NCH3 * 3 + j)
        plsc.subcore_barrier()
        _copy_slab(si, acc, out_hbm.at[ci])

    return k(hs, src_g, dst_s, zeros_nd)


def _rowblk(i):
    return (i, 0)


def _fixed(i):
    return (0, 0)


def _prescale_tc(feature, deg0, deg1):
    def body(f_ref, d0_ref, d1_ref, o_ref):
        n_out = lax.rsqrt(jnp.maximum(d0_ref[:, 0:1] + d1_ref[:, 0:1], 1.0))
        o_ref[...] = f_ref[...] * n_out

    return pl.pallas_call(
        body,
        grid=(_G,),
        in_specs=[
            pl.BlockSpec((_ROWBLK, _D), _rowblk),
            pl.BlockSpec((_ROWBLK, _D), _rowblk),
            pl.BlockSpec((_ROWBLK, _D), _rowblk),
        ],
        out_specs=pl.BlockSpec((_ROWBLK, _D), _rowblk),
        out_shape=jax.ShapeDtypeStruct((_N, _D), jnp.float32),
    )(feature, deg0, deg1)


def _layer_tc(a0, a1, deg0, deg1, W, bvec, ln_g, ln_b,
              apply_ln, emit_hs, emit_colsq):
    def body(a0_ref, a1_ref, d0_ref, d1_ref, w_ref,
             b_ref, g_ref, be_ref, *out_refs):
        n_in = lax.rsqrt(
            jnp.maximum(d0_ref[:, 64:65] + d1_ref[:, 64:65], 1.0))
        g = (a0_ref[...] + a1_ref[...]) * n_in
        h = jnp.dot(g, w_ref[...], preferred_element_type=jnp.float32)
        h = h + b_ref[...]
        if apply_ln:
            mu = jnp.mean(h, axis=-1, keepdims=True)
            var = jnp.mean((h - mu) ** 2, axis=-1, keepdims=True)
            h = (h - mu) * lax.rsqrt(var + 1e-5) * g_ref[...] + be_ref[...]
            h = jnp.maximum(h, 0.0)
        idx = 0
        out_refs[idx][...] = h
        idx += 1
        if emit_hs:
            n_out = lax.rsqrt(
                jnp.maximum(d0_ref[:, 0:1] + d1_ref[:, 0:1], 1.0))
            out_refs[idx][...] = h * n_out
            idx += 1
        if emit_colsq:
            csq = out_refs[idx]

            @pl.when(pl.program_id(0) == 0)
            def _():
                csq[...] = jnp.zeros_like(csq)

            csq[...] += jnp.sum(h * h, axis=0, keepdims=True)

    out_shapes = [jax.ShapeDtypeStruct((_N, _D), jnp.float32)]
    out_specs = [pl.BlockSpec((_ROWBLK, _D), _rowblk)]
    if emit_hs:
        out_shapes.append(jax.ShapeDtypeStruct((_N, _D), jnp.float32))
        out_specs.append(pl.BlockSpec((_ROWBLK, _D), _rowblk))
    if emit_colsq:
        out_shapes.append(jax.ShapeDtypeStruct((1, _D), jnp.float32))
        out_specs.append(pl.BlockSpec((1, _D), _fixed))

    return pl.pallas_call(
        body,
        grid=(_G,),
        in_specs=[
            pl.BlockSpec((_ROWBLK, _D), _rowblk),
            pl.BlockSpec((_ROWBLK, _D), _rowblk),
            pl.BlockSpec((_ROWBLK, _D), _rowblk),
            pl.BlockSpec((_ROWBLK, _D), _rowblk),
            pl.BlockSpec((_D, _D), _fixed),
            pl.BlockSpec((1, _D), _fixed),
            pl.BlockSpec((1, _D), _fixed),
            pl.BlockSpec((1, _D), _fixed),
        ],
        out_specs=out_specs,
        out_shape=out_shapes,
    )(a0, a1, deg0, deg1, W, bvec, ln_g, ln_b)


def _proj_tc(emb, W_tp, b_tp):
    def body(e_ref, w_ref, b_ref, x_ref, csq_ref):
        x = jnp.dot(e_ref[...], w_ref[...], preferred_element_type=jnp.float32)
        x = x + b_ref[...]
        x_ref[...] = x

        @pl.when(pl.program_id(0) == 0)
        def _():
            csq_ref[...] = jnp.zeros_like(csq_ref)

        csq_ref[...] += jnp.sum(x * x, axis=0, keepdims=True)

    return pl.pallas_call(
        body,
        grid=(_G,),
        in_specs=[
            pl.BlockSpec((_ROWBLK, _DL), _rowblk),
            pl.BlockSpec((_DL, _D), _fixed),
            pl.BlockSpec((1, _D), _fixed),
        ],
        out_specs=[
            pl.BlockSpec((_ROWBLK, _D), _rowblk),
            pl.BlockSpec((1, _D), _fixed),
        ],
        out_shape=[
            jax.ShapeDtypeStruct((_N, _D), jnp.float32),
            jax.ShapeDtypeStruct((1, _D), jnp.float32),
        ],
    )(emb, W_tp, b_tp)


def _clip_tc(x, s, s_hat, feature, csq_s, csq_x, params):
    def body(x_ref, s_ref, sh_ref, f_ref, cs_ref, cx_ref, p_ref, score_ref,
             lsum_ref):
        ls = jnp.exp(p_ref[0, 0])
        pre_nz = p_ref[0, 1] != 0.0
        scale = lax.rsqrt(cs_ref[...] * cx_ref[...]) * ls
        xs = x_ref[...] * scale
        logits = lax.dot_general(
            xs, s_ref[...], (((1,), (1,)), ((), ())),
            preferred_element_type=jnp.float32)
        rows = lax.broadcasted_iota(jnp.int32, (_B, _B), 0)
        cols = lax.broadcasted_iota(jnp.int32, (_B, _B), 1)
        eye = rows == cols
        m_row = jnp.max(logits, axis=1, keepdims=True)
        lse_row = m_row + jnp.log(
            jnp.sum(jnp.exp(logits - m_row), axis=1, keepdims=True))
        m_col = jnp.max(logits, axis=0, keepdims=True)
        lse_col = m_col + jnp.log(
            jnp.sum(jnp.exp(logits - m_col), axis=0, keepdims=True))
        diag = jnp.sum(jnp.where(eye, logits, 0.0), axis=1, keepdims=True)
        lse_col_t = jnp.sum(jnp.where(eye, lse_col, 0.0), axis=1,
                            keepdims=True)
        loss_clip = 0.5 * (lse_row + lse_col_t - 2.0 * diag)
        diff = sh_ref[...] - f_ref[...]
        loss_anom = jnp.sqrt(jnp.sum(diff * diff, axis=1, keepdims=True))
        loss = jnp.where(pre_nz, loss_clip, loss_anom + 0.5 * loss_clip)
        mn = jnp.min(loss)
        mx = jnp.max(loss)
        score_ref[...] = (loss - mn) / (mx - mn)

        @pl.when(pl.program_id(0) == 0)
        def _():
            lsum_ref[0, 0] = 0.0

        lsum_ref[0, 0] += jnp.sum(loss)

    return pl.pallas_call(
        body,
        grid=(_NB,),
        in_specs=[
            pl.BlockSpec((_B, _D), _rowblk),
            pl.BlockSpec((_B, _D), _rowblk),
            pl.BlockSpec((_B, _D), _rowblk),
            pl.BlockSpec((_B, _D), _rowblk),
            pl.BlockSpec((1, _D), _fixed),
            pl.BlockSpec((1, _D), _fixed),
            pl.BlockSpec(memory_space=pltpu.SMEM),
        ],
        out_specs=[
            pl.BlockSpec((_B, 1), _rowblk),
            pl.BlockSpec(memory_space=pltpu.SMEM),
        ],
        out_shape=[
            jax.ShapeDtypeStruct((_N, 1), jnp.float32),
            jax.ShapeDtypeStruct((1, 1), jnp.float32),
        ],
    )(x, s, s_hat, feature, csq_s, csq_x, params)


def kernel(feature, embeddings, W_enc, b_enc, ln_g_enc, ln_b_enc, W_dec,
           b_dec, ln_g_dec, ln_b_dec, W_tp, b_tp, logit_scale, edge_index,
           pre, batch_size):
    del batch_size

    src = edge_index[0]
    dst = edge_index[1]
    pad = _EPT_PAD - _EPT
    src_t = src.reshape(_NW, _EPT)
    dst_t = dst.reshape(_NW, _EPT)
    padN = jnp.full((_NW, pad), _N, jnp.int32)
    pad0 = jnp.zeros((_NW, pad), jnp.int32)
    src_gather = jnp.concatenate([src_t, pad0], 1).reshape(_NW, _NCH, _KC)
    pad_d = _NCHD * _KD - _EPT
    padN_d = jnp.full((_NW, pad_d), _N, jnp.int32)
    src_deg = jnp.concatenate([src_t, padN_d], 1).reshape(_NW, _NCHD, _KD)
    dst_deg = jnp.concatenate([dst_t, padN_d], 1).reshape(_NW, _NCHD, _KD)
    dst_spmm = jnp.concatenate([dst_t, padN], 1).reshape(_NW, _NCH, _KC)

    zeros_nd = jnp.zeros((_N, _D), jnp.float32)
    lane = jnp.arange(_D, dtype=jnp.int32)[None, :]
    ones_s = jnp.broadcast_to((lane == 0).astype(jnp.float32), (_KD, _D))
    ones_d = jnp.broadcast_to((lane == 64).astype(jnp.float32), (_KD, _D))

    deg = _degrees_sc(src_deg, dst_deg, zeros_nd, ones_s, ones_d)
    deg0, deg1 = deg[0], deg[1]

    b_enc2 = b_enc.reshape(_L, 1, _D)
    b_dec2 = b_dec.reshape(_L, 1, _D)
    lng_e = ln_g_enc.reshape(1, _D)
    lnb_e = ln_b_enc.reshape(1, _D)
    lng_d = ln_g_dec.reshape(1, _D)
    lnb_d = ln_b_dec.reshape(1, _D)

    hs = _prescale_tc(feature, deg0, deg1)
    agg = _spmm_sc(hs, src_gather, dst_spmm, zeros_nd)
    h1, hs1 = _layer_tc(agg[0], agg[1], deg0, deg1,
                        W_enc[0], b_enc2[0], lng_e, lnb_e,
                        apply_ln=True, emit_hs=True, emit_colsq=False)
    agg = _spmm_sc(hs1, src_gather, dst_spmm, zeros_nd)
    s, hs2, csq_s = _layer_tc(agg[0], agg[1], deg0, deg1,
                              W_enc[1], b_enc2[1], lng_e, lnb_e,
                              apply_ln=False, emit_hs=True, emit_colsq=True)
    agg = _spmm_sc(hs2, src_gather, dst_spmm, zeros_nd)
    h3, hs3 = _layer_tc(agg[0], agg[1], deg0, deg1,
                        W_dec[0], b_dec2[0], lng_d, lnb_d,
                        apply_ln=True, emit_hs=True, emit_colsq=False)
    agg = _spmm_sc(hs3, src_gather, dst_spmm, zeros_nd)
    (s_hat,) = _layer_tc(agg[0], agg[1], deg0, deg1,
                         W_dec[1], b_dec2[1], lng_d, lnb_d,
                         apply_ln=False, emit_hs=False, emit_colsq=False)

    x, csq_x = _proj_tc(embeddings, W_tp, b_tp.reshape(1, _D))
    params = jnp.stack([
        jnp.asarray(logit_scale, jnp.float32).reshape(()),
        jnp.asarray(pre, jnp.float32).reshape(()),
    ]).reshape(1, 2)
    score2d, lsum = _clip_tc(x, s, s_hat, feature, csq_s, csq_x, params)
    return (score2d[:, 0], lsum[0, 0])


# speedup vs baseline: 1.2881x; 1.2881x over previous
import functools

import jax
import jax.numpy as jnp
from jax import lax
from jax.experimental import pallas as pl
from jax.experimental.pallas import tpu as pltpu
from jax.experimental.pallas import tpu_sc as plsc

_N = 10000
_E = 320000
_D = 128
_DL = 512
_B = 2000
_NB = _N // _B
_L = 2

_NC = 2
_NS = 16
_NW = _NC * _NS
_EPT = _E // _NW
_KC = 128
_NCHS = 80
_EPT_PAD = _NCHS * _KC
_KD = 256
_NCHD = 40
_NP = _N + 8
_SLAB = 624
_REM = _N - _NS * _SLAB

_DEGW = 16

_ROWBLK = 1000
_G = _N // _ROWBLK


def _sc_mesh():
    return plsc.VectorSubcoreMesh(core_axis_name="c", subcore_axis_name="s")


def _copy_slab(si, src_ref, dst_ref):
    pltpu.sync_copy(src_ref.at[pl.ds(si * _SLAB, _SLAB)],
                    dst_ref.at[pl.ds(si * _SLAB, _SLAB)])

    @pl.when(si == _NS - 1)
    def _():
        pltpu.sync_copy(src_ref.at[pl.ds(_NS * _SLAB, _REM)],
                        dst_ref.at[pl.ds(_NS * _SLAB, _REM)])


def _degrees_sc(src_deg, dst_pad, zeros_nd, ones_s, ones_d):
    @functools.partial(
        pl.kernel,
        out_type=jax.ShapeDtypeStruct((_NC, _N, _D), jnp.float32),
        mesh=_sc_mesh(),
        scratch_types=[
            pltpu.VMEM((_KD,), jnp.int32),
            pltpu.VMEM((_KD,), jnp.int32),
            pltpu.VMEM((_KD, _D), jnp.float32),
            pltpu.VMEM_SHARED((_NP, _D), jnp.float32),
            pltpu.SemaphoreType.DMA,
            pltpu.SemaphoreType.DMA,
        ],
    )
    def k(src_hbm, dst_hbm, z_hbm, ones_s_hbm, ones_d_hbm, out_hbm, idx_a,
          idx_b, ones_v, acc, sem_a, sem_b):
        ci = lax.axis_index("c")
        si = lax.axis_index("s")
        wid = si * _NC + ci
        _copy_slab(si, z_hbm, acc)

        def phase(e_hbm, carry0):
            def body(i, carry):
                g0 = i * 2
                g1 = g0 + 1
                cp_a = pltpu.async_copy(e_hbm.at[wid, g0], idx_a, sem_a)
                cp_b = pltpu.async_copy(e_hbm.at[wid, g1], idx_b, sem_b)
                cp_a.wait()
                pltpu.sync_copy(ones_v, acc.at[idx_a], add=True)
                cp_b.wait()
                pltpu.sync_copy(ones_v, acc.at[idx_b], add=True)
                return carry

            lax.fori_loop(0, _NCHD // 2, body, carry0)

        pltpu.sync_copy(ones_s_hbm, ones_v)
        plsc.subcore_barrier()
        phase(src_hbm, 0)
        pltpu.sync_copy(ones_d_hbm, ones_v)
        phase(dst_hbm, 0)
        plsc.subcore_barrier()
        _copy_slab(si, acc, out_hbm.at[ci])

    return k(src_deg, dst_pad, zeros_nd, ones_s, ones_d)


def _spmm_sc(hs, src_g, dst_s, zeros_nd):
    @functools.partial(
        pl.kernel,
        out_type=jax.ShapeDtypeStruct((_NC, _N, _D), jnp.float32),
        mesh=_sc_mesh(),
        scratch_types=[
            pltpu.VMEM((_NCHS, _KC), jnp.int32),
            pltpu.VMEM((_KC,), jnp.int32),
            pltpu.VMEM((_KC,), jnp.int32),
            pltpu.VMEM((_KC, _D), jnp.float32),
            pltpu.VMEM((_KC, _D), jnp.float32),
            pltpu.VMEM_SHARED((_NP, _D), jnp.float32),
            pltpu.SemaphoreType.DMA,
            pltpu.SemaphoreType.DMA,
            pltpu.SemaphoreType.DMA,
            pltpu.SemaphoreType.DMA,
            pltpu.SemaphoreType.DMA,
            pltpu.SemaphoreType.DMA,
        ],
    )
    def k(hs_hbm, src_hbm, dst_hbm, z_hbm, out_hbm, sidx, didx_a, didx_b,
          rows_a, rows_b, acc, sem_a, sem_b, sem_da, sem_db, sem_wa,
          sem_wb):
        ci = lax.axis_index("c")
        si = lax.axis_index("s")
        wid = si * _NC + ci
        _copy_slab(si, z_hbm, acc)
        pltpu.sync_copy(src_hbm.at[wid], sidx)
        plsc.subcore_barrier()

        def _scat_wait(sem):
            pltpu.make_async_copy(rows_a, acc.at[didx_a], sem).wait()

        def body(i, carry):
            @pl.when(i > 0)
            def _():
                _scat_wait(sem_wa)
                _scat_wait(sem_wb)

            g0 = i * 2
            g1 = g0 + 1
            cp_a = pltpu.async_copy(hs_hbm.at[sidx.at[g0]], rows_a, sem_a)
            cp_da = pltpu.async_copy(dst_hbm.at[wid, g0], didx_a, sem_da)
            cp_b = pltpu.async_copy(hs_hbm.at[sidx.at[g1]], rows_b, sem_b)
            cp_db = pltpu.async_copy(dst_hbm.at[wid, g1], didx_b, sem_db)
            cp_a.wait()
            cp_da.wait()
            pltpu.async_copy(rows_a, acc.at[didx_a], sem_wa, add=True)
            cp_b.wait()
            cp_db.wait()
            pltpu.async_copy(rows_b, acc.at[didx_b], sem_wb, add=True)
            return carry

        lax.fori_loop(0, _NCHS // 2, body, 0)
        _scat_wait(sem_wa)
        _scat_wait(sem_wb)
        plsc.subcore_barrier()
        _copy_slab(si, acc, out_hbm.at[ci])

    return k(hs, src_g, dst_s, zeros_nd)


def _rowblk(i):
    return (i, 0)


def _fixed(i):
    return (0, 0)


def _prescale_tc(feature, deg0, deg1):
    def body(f_ref, d0_ref, d1_ref, o_ref):
        n_out = lax.rsqrt(jnp.maximum(d0_ref[:, 0:1] + d1_ref[:, 0:1], 1.0))
        o_ref[...] = f_ref[...] * n_out

    return pl.pallas_call(
        body,
        grid=(_G,),
        in_specs=[
            pl.BlockSpec((_ROWBLK, _D), _rowblk),
            pl.BlockSpec((_ROWBLK, _D), _rowblk),
            pl.BlockSpec((_ROWBLK, _D), _rowblk),
        ],
        out_specs=pl.BlockSpec((_ROWBLK, _D), _rowblk),
        out_shape=jax.ShapeDtypeStruct((_N, _D), jnp.float32),
    )(feature, deg0, deg1)


def _layer_tc(a0, a1, deg0, deg1, W, bvec, ln_g, ln_b,
              apply_ln, emit_hs, emit_colsq):
    def body(a0_ref, a1_ref, d0_ref, d1_ref, w_ref,
             b_ref, g_ref, be_ref, *out_refs):
        n_in = lax.rsqrt(
            jnp.maximum(d0_ref[:, 64:65] + d1_ref[:, 64:65], 1.0))
        g = (a0_ref[...] + a1_ref[...]) * n_in
        h = jnp.dot(g, w_ref[...], preferred_element_type=jnp.float32)
        h = h + b_ref[...]
        if apply_ln:
            mu = jnp.mean(h, axis=-1, keepdims=True)
            var = jnp.mean((h - mu) ** 2, axis=-1, keepdims=True)
            h = (h - mu) * lax.rsqrt(var + 1e-5) * g_ref[...] + be_ref[...]
            h = jnp.maximum(h, 0.0)
        idx = 0
        out_refs[idx][...] = h
        idx += 1
        if emit_hs:
            n_out = lax.rsqrt(
                jnp.maximum(d0_ref[:, 0:1] + d1_ref[:, 0:1], 1.0))
            out_refs[idx][...] = h * n_out
            idx += 1
        if emit_colsq:
            csq = out_refs[idx]

            @pl.when(pl.program_id(0) == 0)
            def _():
                csq[...] = jnp.zeros_like(csq)

            csq[...] += jnp.sum(h * h, axis=0, keepdims=True)

    out_shapes = [jax.ShapeDtypeStruct((_N, _D), jnp.float32)]
    out_specs = [pl.BlockSpec((_ROWBLK, _D), _rowblk)]
    if emit_hs:
        out_shapes.append(jax.ShapeDtypeStruct((_N, _D), jnp.float32))
        out_specs.append(pl.BlockSpec((_ROWBLK, _D), _rowblk))
    if emit_colsq:
        out_shapes.append(jax.ShapeDtypeStruct((1, _D), jnp.float32))
        out_specs.append(pl.BlockSpec((1, _D), _fixed))

    return pl.pallas_call(
        body,
        grid=(_G,),
        in_specs=[
            pl.BlockSpec((_ROWBLK, _D), _rowblk),
            pl.BlockSpec((_ROWBLK, _D), _rowblk),
            pl.BlockSpec((_ROWBLK, _D), _rowblk),
            pl.BlockSpec((_ROWBLK, _D), _rowblk),
            pl.BlockSpec((_D, _D), _fixed),
            pl.BlockSpec((1, _D), _fixed),
            pl.BlockSpec((1, _D), _fixed),
            pl.BlockSpec((1, _D), _fixed),
        ],
        out_specs=out_specs,
        out_shape=out_shapes,
    )(a0, a1, deg0, deg1, W, bvec, ln_g, ln_b)


def _proj_tc(emb, W_tp, b_tp):
    def body(e_ref, w_ref, b_ref, x_ref, csq_ref):
        x = jnp.dot(e_ref[...], w_ref[...], preferred_element_type=jnp.float32)
        x = x + b_ref[...]
        x_ref[...] = x

        @pl.when(pl.program_id(0) == 0)
        def _():
            csq_ref[...] = jnp.zeros_like(csq_ref)

        csq_ref[...] += jnp.sum(x * x, axis=0, keepdims=True)

    return pl.pallas_call(
        body,
        grid=(_G,),
        in_specs=[
            pl.BlockSpec((_ROWBLK, _DL), _rowblk),
            pl.BlockSpec((_DL, _D), _fixed),
            pl.BlockSpec((1, _D), _fixed),
        ],
        out_specs=[
            pl.BlockSpec((_ROWBLK, _D), _rowblk),
            pl.BlockSpec((1, _D), _fixed),
        ],
        out_shape=[
            jax.ShapeDtypeStruct((_N, _D), jnp.float32),
            jax.ShapeDtypeStruct((1, _D), jnp.float32),
        ],
    )(emb, W_tp, b_tp)


def _clip_tc(x, s, s_hat, feature, csq_s, csq_x, params):
    def body(x_ref, s_ref, sh_ref, f_ref, cs_ref, cx_ref, p_ref, score_ref,
             lsum_ref):
        ls = jnp.exp(p_ref[0, 0])
        pre_nz = p_ref[0, 1] != 0.0
        scale = lax.rsqrt(cs_ref[...] * cx_ref[...]) * ls
        xs = x_ref[...] * scale
        logits = lax.dot_general(
            xs, s_ref[...], (((1,), (1,)), ((), ())),
            preferred_element_type=jnp.float32)
        rows = lax.broadcasted_iota(jnp.int32, (_B, _B), 0)
        cols = lax.broadcasted_iota(jnp.int32, (_B, _B), 1)
        eye = rows == cols
        m_row = jnp.max(logits, axis=1, keepdims=True)
        lse_row = m_row + jnp.log(
            jnp.sum(jnp.exp(logits - m_row), axis=1, keepdims=True))
        m_col = jnp.max(logits, axis=0, keepdims=True)
        lse_col = m_col + jnp.log(
            jnp.sum(jnp.exp(logits - m_col), axis=0, keepdims=True))
        diag = jnp.sum(jnp.where(eye, logits, 0.0), axis=1, keepdims=True)
        lse_col_t = jnp.sum(jnp.where(eye, lse_col, 0.0), axis=1,
                            keepdims=True)
        loss_clip = 0.5 * (lse_row + lse_col_t - 2.0 * diag)
        diff = sh_ref[...] - f_ref[...]
        loss_anom = jnp.sqrt(jnp.sum(diff * diff, axis=1, keepdims=True))
        loss = jnp.where(pre_nz, loss_clip, loss_anom + 0.5 * loss_clip)
        mn = jnp.min(loss)
        mx = jnp.max(loss)
        score_ref[...] = (loss - mn) / (mx - mn)

        @pl.when(pl.program_id(0) == 0)
        def _():
            lsum_ref[0, 0] = 0.0

        lsum_ref[0, 0] += jnp.sum(loss)

    return pl.pallas_call(
        body,
        grid=(_NB,),
        in_specs=[
            pl.BlockSpec((_B, _D), _rowblk),
            pl.BlockSpec((_B, _D), _rowblk),
            pl.BlockSpec((_B, _D), _rowblk),
            pl.BlockSpec((_B, _D), _rowblk),
            pl.BlockSpec((1, _D), _fixed),
            pl.BlockSpec((1, _D), _fixed),
            pl.BlockSpec(memory_space=pltpu.SMEM),
        ],
        out_specs=[
            pl.BlockSpec((_B, 1), _rowblk),
            pl.BlockSpec(memory_space=pltpu.SMEM),
        ],
        out_shape=[
            jax.ShapeDtypeStruct((_N, 1), jnp.float32),
            jax.ShapeDtypeStruct((1, 1), jnp.float32),
        ],
    )(x, s, s_hat, feature, csq_s, csq_x, params)


def kernel(feature, embeddings, W_enc, b_enc, ln_g_enc, ln_b_enc, W_dec,
           b_dec, ln_g_dec, ln_b_dec, W_tp, b_tp, logit_scale, edge_index,
           pre, batch_size):
    del batch_size

    src = edge_index[0]
    dst = edge_index[1]
    pad = _EPT_PAD - _EPT
    src_t = src.reshape(_NW, _EPT)
    dst_t = dst.reshape(_NW, _EPT)
    padN = jnp.full((_NW, pad), _N, jnp.int32)
    pad0 = jnp.zeros((_NW, pad), jnp.int32)
    src_gather = jnp.concatenate([src_t, pad0], 1).reshape(_NW, _NCHS, _KC)
    pad_d = _NCHD * _KD - _EPT
    padN_d = jnp.full((_NW, pad_d), _N, jnp.int32)
    src_deg = jnp.concatenate([src_t, padN_d], 1).reshape(_NW, _NCHD, _KD)
    dst_deg = jnp.concatenate([dst_t, padN_d], 1).reshape(_NW, _NCHD, _KD)
    dst_spmm = jnp.concatenate([dst_t, padN], 1).reshape(_NW, _NCHS, _KC)

    zeros_nd = jnp.zeros((_N, _D), jnp.float32)
    lane = jnp.arange(_D, dtype=jnp.int32)[None, :]
    ones_s = jnp.broadcast_to((lane == 0).astype(jnp.float32), (_KD, _D))
    ones_d = jnp.broadcast_to((lane == 64).astype(jnp.float32), (_KD, _D))

    deg = _degrees_sc(src_deg, dst_deg, zeros_nd, ones_s, ones_d)
    deg0, deg1 = deg[0], deg[1]

    b_enc2 = b_enc.reshape(_L, 1, _D)
    b_dec2 = b_dec.reshape(_L, 1, _D)
    lng_e = ln_g_enc.reshape(1, _D)
    lnb_e = ln_b_enc.reshape(1, _D)
    lng_d = ln_g_dec.reshape(1, _D)
    lnb_d = ln_b_dec.reshape(1, _D)

    hs = _prescale_tc(feature, deg0, deg1)
    agg = _spmm_sc(hs, src_gather, dst_spmm, zeros_nd)
    h1, hs1 = _layer_tc(agg[0], agg[1], deg0, deg1,
                        W_enc[0], b_enc2[0], lng_e, lnb_e,
                        apply_ln=True, emit_hs=True, emit_colsq=False)
    agg = _spmm_sc(hs1, src_gather, dst_spmm, zeros_nd)
    s, hs2, csq_s = _layer_tc(agg[0], agg[1], deg0, deg1,
                              W_enc[1], b_enc2[1], lng_e, lnb_e,
                              apply_ln=False, emit_hs=True, emit_colsq=True)
    agg = _spmm_sc(hs2, src_gather, dst_spmm, zeros_nd)
    h3, hs3 = _layer_tc(agg[0], agg[1], deg0, deg1,
                        W_dec[0], b_dec2[0], lng_d, lnb_d,
                        apply_ln=True, emit_hs=True, emit_colsq=False)
    agg = _spmm_sc(hs3, src_gather, dst_spmm, zeros_nd)
    (s_hat,) = _layer_tc(agg[0], agg[1], deg0, deg1,
                         W_dec[1], b_dec2[1], lng_d, lnb_d,
                         apply_ln=False, emit_hs=False, emit_colsq=False)

    x, csq_x = _proj_tc(embeddings, W_tp, b_tp.reshape(1, _D))
    params = jnp.stack([
        jnp.asarray(logit_scale, jnp.float32).reshape(()),
        jnp.asarray(pre, jnp.float32).reshape(()),
    ]).reshape(1, 2)
    score2d, lsum = _clip_tc(x, s, s_hat, feature, csq_s, csq_x, params)
    return (score2d[:, 0], lsum[0, 0])


# speedup vs baseline: 1.2891x; 1.0008x over previous
import functools

import jax
import jax.numpy as jnp
from jax import lax
from jax.experimental import pallas as pl
from jax.experimental.pallas import tpu as pltpu
from jax.experimental.pallas import tpu_sc as plsc

_N = 10000
_E = 320000
_D = 128
_DL = 512
_B = 2000
_NB = _N // _B
_L = 2

_NC = 2
_NS = 16
_NW = _NC * _NS
_EPT = _E // _NW
_KC = 128
_NCHS = 80
_EPT_PAD = _NCHS * _KC
_KD = 256
_NCHD = 40
_NP = _N + 8
_SLAB = 624
_REM = _N - _NS * _SLAB

_DEGW = 16

_ROWBLK = 1000
_G = _N // _ROWBLK


def _sc_mesh():
    return plsc.VectorSubcoreMesh(core_axis_name="c", subcore_axis_name="s")


def _copy_slab(si, src_ref, dst_ref):
    pltpu.sync_copy(src_ref.at[pl.ds(si * _SLAB, _SLAB)],
                    dst_ref.at[pl.ds(si * _SLAB, _SLAB)])

    @pl.when(si == _NS - 1)
    def _():
        pltpu.sync_copy(src_ref.at[pl.ds(_NS * _SLAB, _REM)],
                        dst_ref.at[pl.ds(_NS * _SLAB, _REM)])


def _degrees_sc(src_deg, dst_pad, zeros_nd, ones_s, ones_d):
    @functools.partial(
        pl.kernel,
        out_type=jax.ShapeDtypeStruct((_NC, _N, _D), jnp.float32),
        mesh=_sc_mesh(),
        scratch_types=[
            pltpu.VMEM((_KD,), jnp.int32),
            pltpu.VMEM((_KD,), jnp.int32),
            pltpu.VMEM((_KD, _D), jnp.float32),
            pltpu.VMEM_SHARED((_NP, _D), jnp.float32),
            pltpu.SemaphoreType.DMA,
            pltpu.SemaphoreType.DMA,
        ],
    )
    def k(src_hbm, dst_hbm, z_hbm, ones_s_hbm, ones_d_hbm, out_hbm, idx_a,
          idx_b, ones_v, acc, sem_a, sem_b):
        ci = lax.axis_index("c")
        si = lax.axis_index("s")
        wid = si * _NC + ci
        _copy_slab(si, z_hbm, acc)

        def phase(e_hbm, carry0):
            def body(i, carry):
                g0 = i * 2
                g1 = g0 + 1
                cp_a = pltpu.async_copy(e_hbm.at[wid, g0], idx_a, sem_a)
                cp_b = pltpu.async_copy(e_hbm.at[wid, g1], idx_b, sem_b)
                cp_a.wait()
                pltpu.sync_copy(ones_v, acc.at[idx_a], add=True)
                cp_b.wait()
                pltpu.sync_copy(ones_v, acc.at[idx_b], add=True)
                return carry

            lax.fori_loop(0, _NCHD // 2, body, carry0)

        pltpu.sync_copy(ones_s_hbm, ones_v)
        plsc.subcore_barrier()
        phase(src_hbm, 0)
        pltpu.sync_copy(ones_d_hbm, ones_v)
        phase(dst_hbm, 0)
        plsc.subcore_barrier()
        _copy_slab(si, acc, out_hbm.at[ci])

    return k(src_deg, dst_pad, zeros_nd, ones_s, ones_d)


def _spmm_sc(hs, src_g, dst_s, zeros_nd):
    @functools.partial(
        pl.kernel,
        out_type=jax.ShapeDtypeStruct((_NC, _N, _D), jnp.float32),
        mesh=_sc_mesh(),
        scratch_types=[
            pltpu.VMEM((_NCHS, _KC), jnp.int32),
            pltpu.VMEM((_KC,), jnp.int32),
            pltpu.VMEM((_KC,), jnp.int32),
            pltpu.VMEM((_KC, _D), jnp.float32),
            pltpu.VMEM((_KC, _D), jnp.float32),
            pltpu.VMEM_SHARED((_NP, _D), jnp.float32),
            pltpu.SemaphoreType.DMA,
            pltpu.SemaphoreType.DMA,
            pltpu.SemaphoreType.DMA,
            pltpu.SemaphoreType.DMA,
            pltpu.SemaphoreType.DMA,
            pltpu.SemaphoreType.DMA,
        ],
    )
    def k(hs_hbm, src_hbm, dst_hbm, z_hbm, out_hbm, sidx, didx_a, didx_b,
          rows_a, rows_b, acc, sem_a, sem_b, sem_da, sem_db, sem_wa,
          sem_wb):
        ci = lax.axis_index("c")
        si = lax.axis_index("s")
        wid = si * _NC + ci
        _copy_slab(si, z_hbm, acc)
        pltpu.sync_copy(src_hbm.at[wid], sidx)
        plsc.subcore_barrier()

        def _scat_wait(sem):
            pltpu.make_async_copy(rows_a, acc.at[didx_a], sem).wait()

        def body(i, carry):
            @pl.when(i > 0)
            def _():
                _scat_wait(sem_wa)
                _scat_wait(sem_wb)

            g0 = i * 2
            g1 = g0 + 1
            cp_a = pltpu.async_copy(hs_hbm.at[sidx.at[g0]], rows_a, sem_a)
            cp_da = pltpu.async_copy(dst_hbm.at[wid, g0], didx_a, sem_da)
            cp_b = pltpu.async_copy(hs_hbm.at[sidx.at[g1]], rows_b, sem_b)
            cp_db = pltpu.async_copy(dst_hbm.at[wid, g1], didx_b, sem_db)
            cp_a.wait()
            cp_da.wait()
            pltpu.async_copy(rows_a, acc.at[didx_a], sem_wa, add=True)
            cp_b.wait()
            cp_db.wait()
            pltpu.async_copy(rows_b, acc.at[didx_b], sem_wb, add=True)
            return carry

        lax.fori_loop(0, _NCHS // 2, body, 0)
        _scat_wait(sem_wa)
        _scat_wait(sem_wb)
        plsc.subcore_barrier()
        _copy_slab(si, acc, out_hbm.at[ci])

    return k(hs, src_g, dst_s, zeros_nd)


def _rowblk(i):
    return (i, 0)


def _fixed(i):
    return (0, 0)


def _prescale_tc(feature, deg0, deg1):
    def body(f_ref, d0_ref, d1_ref, o_ref):
        n_out = lax.rsqrt(jnp.maximum(d0_ref[:, 0:1] + d1_ref[:, 0:1], 1.0))
        o_ref[...] = f_ref[...] * n_out

    return pl.pallas_call(
        body,
        grid=(_G,),
        in_specs=[
            pl.BlockSpec((_ROWBLK, _D), _rowblk),
            pl.BlockSpec((_ROWBLK, _D), _rowblk),
            pl.BlockSpec((_ROWBLK, _D), _rowblk),
        ],
        out_specs=pl.BlockSpec((_ROWBLK, _D), _rowblk),
        out_shape=jax.ShapeDtypeStruct((_N, _D), jnp.float32),
    )(feature, deg0, deg1)


def _layer_tc(a0, a1, deg0, deg1, W, bvec, ln_g, ln_b,
              apply_ln, emit_hs, emit_colsq, emit_h=True):
    def body(a0_ref, a1_ref, d0_ref, d1_ref, w_ref,
             b_ref, g_ref, be_ref, *out_refs):
        n_in = lax.rsqrt(
            jnp.maximum(d0_ref[:, 64:65] + d1_ref[:, 64:65], 1.0))
        g = (a0_ref[...] + a1_ref[...]) * n_in
        h = jnp.dot(g, w_ref[...], preferred_element_type=jnp.float32)
        h = h + b_ref[...]
        if apply_ln:
            mu = jnp.mean(h, axis=-1, keepdims=True)
            var = jnp.mean((h - mu) ** 2, axis=-1, keepdims=True)
            h = (h - mu) * lax.rsqrt(var + 1e-5) * g_ref[...] + be_ref[...]
            h = jnp.maximum(h, 0.0)
        idx = 0
        if emit_h:
            out_refs[idx][...] = h
            idx += 1
        if emit_hs:
            n_out = lax.rsqrt(
                jnp.maximum(d0_ref[:, 0:1] + d1_ref[:, 0:1], 1.0))
            out_refs[idx][...] = h * n_out
            idx += 1
        if emit_colsq:
            csq = out_refs[idx]

            @pl.when(pl.program_id(0) == 0)
            def _():
                csq[...] = jnp.zeros_like(csq)

            csq[...] += jnp.sum(h * h, axis=0, keepdims=True)

    out_shapes = []
    out_specs = []
    if emit_h:
        out_shapes.append(jax.ShapeDtypeStruct((_N, _D), jnp.float32))
        out_specs.append(pl.BlockSpec((_ROWBLK, _D), _rowblk))
    if emit_hs:
        out_shapes.append(jax.ShapeDtypeStruct((_N, _D), jnp.float32))
        out_specs.append(pl.BlockSpec((_ROWBLK, _D), _rowblk))
    if emit_colsq:
        out_shapes.append(jax.ShapeDtypeStruct((1, _D), jnp.float32))
        out_specs.append(pl.BlockSpec((1, _D), _fixed))

    return pl.pallas_call(
        body,
        grid=(_G,),
        in_specs=[
            pl.BlockSpec((_ROWBLK, _D), _rowblk),
            pl.BlockSpec((_ROWBLK, _D), _rowblk),
            pl.BlockSpec((_ROWBLK, _D), _rowblk),
            pl.BlockSpec((_ROWBLK, _D), _rowblk),
            pl.BlockSpec((_D, _D), _fixed),
            pl.BlockSpec((1, _D), _fixed),
            pl.BlockSpec((1, _D), _fixed),
            pl.BlockSpec((1, _D), _fixed),
        ],
        out_specs=out_specs,
        out_shape=out_shapes,
    )(a0, a1, deg0, deg1, W, bvec, ln_g, ln_b)


def _proj_tc(emb, W_tp, b_tp):
    def body(e_ref, w_ref, b_ref, x_ref, csq_ref):
        x = jnp.dot(e_ref[...], w_ref[...], preferred_element_type=jnp.float32)
        x = x + b_ref[...]
        x_ref[...] = x

        @pl.when(pl.program_id(0) == 0)
        def _():
            csq_ref[...] = jnp.zeros_like(csq_ref)

        csq_ref[...] += jnp.sum(x * x, axis=0, keepdims=True)

    return pl.pallas_call(
        body,
        grid=(_G,),
        in_specs=[
            pl.BlockSpec((_ROWBLK, _DL), _rowblk),
            pl.BlockSpec((_DL, _D), _fixed),
            pl.BlockSpec((1, _D), _fixed),
        ],
        out_specs=[
            pl.BlockSpec((_ROWBLK, _D), _rowblk),
            pl.BlockSpec((1, _D), _fixed),
        ],
        out_shape=[
            jax.ShapeDtypeStruct((_N, _D), jnp.float32),
            jax.ShapeDtypeStruct((1, _D), jnp.float32),
        ],
    )(emb, W_tp, b_tp)


def _clip_tc(x, s, s_hat, feature, csq_s, csq_x, params):
    def body(x_ref, s_ref, sh_ref, f_ref, cs_ref, cx_ref, p_ref, score_ref,
             lsum_ref):
        ls = jnp.exp(p_ref[0, 0])
        pre_nz = p_ref[0, 1] != 0.0
        scale = lax.rsqrt(cs_ref[...] * cx_ref[...]) * ls
        xs = x_ref[...] * scale
        logits = lax.dot_general(
            xs, s_ref[...], (((1,), (1,)), ((), ())),
            preferred_element_type=jnp.float32)
        m_row = jnp.max(logits, axis=1, keepdims=True)
        lse_row = m_row + jnp.log(
            jnp.sum(jnp.exp(logits - m_row), axis=1, keepdims=True))
        m_col = jnp.max(logits, axis=0, keepdims=True)
        lse_col = m_col + jnp.log(
            jnp.sum(jnp.exp(logits - m_col), axis=0, keepdims=True))
        diag = jnp.sum(xs * s_ref[...], axis=1, keepdims=True)
        lse_col_t = lse_col.reshape(_B, 1)
        loss_clip = 0.5 * (lse_row + lse_col_t - 2.0 * diag)
        diff = sh_ref[...] - f_ref[...]
        loss_anom = jnp.sqrt(jnp.sum(diff * diff, axis=1, keepdims=True))
        loss = jnp.where(pre_nz, loss_clip, loss_anom + 0.5 * loss_clip)
        mn = jnp.min(loss)
        mx = jnp.max(loss)
        score_ref[...] = (loss - mn) / (mx - mn)

        @pl.when(pl.program_id(0) == 0)
        def _():
            lsum_ref[0, 0] = 0.0

        lsum_ref[0, 0] += jnp.sum(loss)

    return pl.pallas_call(
        body,
        grid=(_NB,),
        in_specs=[
            pl.BlockSpec((_B, _D), _rowblk),
            pl.BlockSpec((_B, _D), _rowblk),
            pl.BlockSpec((_B, _D), _rowblk),
            pl.BlockSpec((_B, _D), _rowblk),
            pl.BlockSpec((1, _D), _fixed),
            pl.BlockSpec((1, _D), _fixed),
            pl.BlockSpec(memory_space=pltpu.SMEM),
        ],
        out_specs=[
            pl.BlockSpec((_B, 1), _rowblk),
            pl.BlockSpec(memory_space=pltpu.SMEM),
        ],
        out_shape=[
            jax.ShapeDtypeStruct((_N, 1), jnp.float32),
            jax.ShapeDtypeStruct((1, 1), jnp.float32),
        ],
    )(x, s, s_hat, feature, csq_s, csq_x, params)


def kernel(feature, embeddings, W_enc, b_enc, ln_g_enc, ln_b_enc, W_dec,
           b_dec, ln_g_dec, ln_b_dec, W_tp, b_tp, logit_scale, edge_index,
           pre, batch_size):
    del batch_size

    src = edge_index[0]
    dst = edge_index[1]
    pad = _EPT_PAD - _EPT
    src_t = src.reshape(_NW, _EPT)
    dst_t = dst.reshape(_NW, _EPT)
    padN = jnp.full((_NW, pad), _N, jnp.int32)
    pad0 = jnp.zeros((_NW, pad), jnp.int32)
    src_gather = jnp.concatenate([src_t, pad0], 1).reshape(_NW, _NCHS, _KC)
    pad_d = _NCHD * _KD - _EPT
    padN_d = jnp.full((_NW, pad_d), _N, jnp.int32)
    src_deg = jnp.concatenate([src_t, padN_d], 1).reshape(_NW, _NCHD, _KD)
    dst_deg = jnp.concatenate([dst_t, padN_d], 1).reshape(_NW, _NCHD, _KD)
    dst_spmm = jnp.concatenate([dst_t, padN], 1).reshape(_NW, _NCHS, _KC)

    zeros_nd = jnp.zeros((_N, _D), jnp.float32)
    lane = jnp.arange(_D, dtype=jnp.int32)[None, :]
    ones_s = jnp.broadcast_to((lane == 0).astype(jnp.float32), (_KD, _D))
    ones_d = jnp.broadcast_to((lane == 64).astype(jnp.float32), (_KD, _D))

    deg = _degrees_sc(src_deg, dst_deg, zeros_nd, ones_s, ones_d)
    deg0, deg1 = deg[0], deg[1]

    b_enc2 = b_enc.reshape(_L, 1, _D)
    b_dec2 = b_dec.reshape(_L, 1, _D)
    lng_e = ln_g_enc.reshape(1, _D)
    lnb_e = ln_b_enc.reshape(1, _D)
    lng_d = ln_g_dec.reshape(1, _D)
    lnb_d = ln_b_dec.reshape(1, _D)

    hs = _prescale_tc(feature, deg0, deg1)
    agg = _spmm_sc(hs, src_gather, dst_spmm, zeros_nd)
    (hs1,) = _layer_tc(agg[0], agg[1], deg0, deg1,
                       W_enc[0], b_enc2[0], lng_e, lnb_e,
                       apply_ln=True, emit_hs=True, emit_colsq=False,
                       emit_h=False)
    agg = _spmm_sc(hs1, src_gather, dst_spmm, zeros_nd)
    s, hs2, csq_s = _layer_tc(agg[0], agg[1], deg0, deg1,
                              W_enc[1], b_enc2[1], lng_e, lnb_e,
                              apply_ln=False, emit_hs=True, emit_colsq=True)
    agg = _spmm_sc(hs2, src_gather, dst_spmm, zeros_nd)
    (hs3,) = _layer_tc(agg[0], agg[1], deg0, deg1,
                       W_dec[0], b_dec2[0], lng_d, lnb_d,
                       apply_ln=True, emit_hs=True, emit_colsq=False,
                       emit_h=False)
    agg = _spmm_sc(hs3, src_gather, dst_spmm, zeros_nd)
    (s_hat,) = _layer_tc(agg[0], agg[1], deg0, deg1,
                         W_dec[1], b_dec2[1], lng_d, lnb_d,
                         apply_ln=False, emit_hs=False, emit_colsq=False)

    x, csq_x = _proj_tc(embeddings, W_tp, b_tp.reshape(1, _D))
    params = jnp.stack([
        jnp.asarray(logit_scale, jnp.float32).reshape(()),
        jnp.asarray(pre, jnp.float32).reshape(()),
    ]).reshape(1, 2)
    score2d, lsum = _clip_tc(x, s, s_hat, feature, csq_s, csq_x, params)
    return (score2d[:, 0], lsum[0, 0])
